# baseline (speedup 1.0000x reference)
import functools

import jax
import jax.numpy as jnp
from jax.experimental import pallas as pl
from jax.experimental.pallas import tpu as pltpu


def _layer_body(lo_ref, hi_ref, src_ref, w_ref, b_ref, out_ref, s_ref, *,
                relu, kh):
    @pl.when(pl.program_id(0) == 0)
    def _():
        s_ref[...] = jnp.dot(src_ref[...], w_ref[...],
                             preferred_element_type=jnp.float32
                             ).astype(jnp.bfloat16)
    acc = jnp.dot(lo_ref[...].astype(jnp.bfloat16), s_ref[:kh, :],
                  preferred_element_type=jnp.float32)
    acc = acc + jnp.dot(hi_ref[...].astype(jnp.bfloat16), s_ref[kh:, :],
                        preferred_element_type=jnp.float32)
    acc = acc + b_ref[...]
    if relu:
        acc = jnp.maximum(acc, 0.0)
    out_ref[...] = acc


def _layer(mat, src, w, b, *, relu, block_rows):
    rows, k = mat.shape
    kf, f = src.shape
    c = w.shape[1]
    kh = k // 2
    grid = (rows // block_rows,)
    return pl.pallas_call(
        functools.partial(_layer_body, relu=relu, kh=kh),
        grid=grid,
        in_specs=[
            pl.BlockSpec((block_rows, kh), lambda i: (i, 0)),
            pl.BlockSpec((block_rows, kh), lambda i: (i, 1)),
            pl.BlockSpec((kf, f), lambda i: (0, 0)),
            pl.BlockSpec((f, c), lambda i: (0, 0)),
            pl.BlockSpec((1, c), lambda i: (0, 0)),
        ],
        out_specs=pl.BlockSpec((block_rows, c), lambda i: (i, 0)),
        out_shape=jax.ShapeDtypeStruct((rows, c), jnp.float32),
        scratch_shapes=[pltpu.VMEM((kf, c), jnp.bfloat16)],
        compiler_params=pltpu.CompilerParams(
            dimension_semantics=("arbitrary",),
        ),
    )(mat, mat, src, w, b)


def kernel(x, NF, FN, W1, b1, W2, b2):
    b1r = b1.reshape(1, -1)
    b2r = b2.reshape(1, -1)
    h = _layer(FN, x, W1, b1r, relu=True, block_rows=128)
    out = _layer(NF, h, W2, b2r, relu=False, block_rows=128)
    return out


# speedup vs baseline: 1.2311x; 1.2311x over previous
import functools

import jax
import jax.numpy as jnp
from jax.experimental import pallas as pl
from jax.experimental.pallas import tpu as pltpu

_DEPTH = 4
_BLOCK = 256


def _gcn_body(fn_ref, nf_ref, x_ref, w1_ref, b1_ref, w2_ref, b2_ref,
              out_ref, buf_ref, s1_ref, s2_ref, h_ref, sem_ref):
    m, _ = fn_ref.shape
    n, _ = nf_ref.shape
    nchunk1 = m // _BLOCK
    nchunk2 = n // _BLOCK
    total = nchunk1 + nchunk2

    def copy_in(c, slot):
        c1i = jnp.minimum(c, nchunk1 - 1)
        c2i = jnp.maximum(c - nchunk1, 0)

        def _fn():
            pltpu.make_async_copy(
                fn_ref.at[pl.ds(c1i * _BLOCK, _BLOCK), :],
                buf_ref.at[slot], sem_ref.at[slot]).start()

        def _nf():
            pltpu.make_async_copy(
                nf_ref.at[pl.ds(c2i * _BLOCK, _BLOCK), :],
                buf_ref.at[slot], sem_ref.at[slot]).start()

        jax.lax.cond(c < nchunk1, _fn, _nf)

    def wait(slot):
        pltpu.make_async_copy(
            fn_ref.at[pl.ds(0, _BLOCK), :],
            buf_ref.at[slot], sem_ref.at[slot]).wait()

    for c in range(_DEPTH):
        copy_in(c, c)
    s1_ref[...] = jnp.dot(x_ref[...], w1_ref[...],
                          preferred_element_type=jnp.float32
                          ).astype(jnp.bfloat16)

    def body1(c, carry):
        slot = jax.lax.rem(c, _DEPTH)
        wait(slot)
        blk = buf_ref[slot].astype(jnp.bfloat16)
        acc = jnp.dot(blk, s1_ref[...], preferred_element_type=jnp.float32)
        h_ref[pl.ds(c * _BLOCK, _BLOCK), :] = jnp.maximum(
            acc + b1_ref[...], 0.0)
        nxt = c + _DEPTH

        @pl.when(nxt < total)
        def _():
            copy_in(nxt, slot)
        return carry

    jax.lax.fori_loop(0, nchunk1, body1, 0)

    s2_ref[...] = jnp.dot(h_ref[...], w2_ref[...],
                          preferred_element_type=jnp.float32
                          ).astype(jnp.bfloat16)

    def body2(c, carry):
        slot = jax.lax.rem(c, _DEPTH)
        wait(slot)
        blk = buf_ref[slot].astype(jnp.bfloat16)
        acc = jnp.dot(blk, s2_ref[...], preferred_element_type=jnp.float32)
        out_ref[pl.ds((c - nchunk1) * _BLOCK, _BLOCK), :] = acc + b2_ref[...]
        nxt = c + _DEPTH

        @pl.when(nxt < total)
        def _():
            copy_in(nxt, slot)
        return carry

    jax.lax.fori_loop(nchunk1, total, body2, 0)


def kernel(x, NF, FN, W1, b1, W2, b2):
    m, k = FN.shape
    n, _ = NF.shape
    kf, f = x.shape
    c1 = W1.shape[1]
    c2 = W2.shape[1]
    return pl.pallas_call(
        _gcn_body,
        in_specs=[
            pl.BlockSpec(memory_space=pltpu.MemorySpace.HBM),
            pl.BlockSpec(memory_space=pltpu.MemorySpace.HBM),
            pl.BlockSpec((kf, f), lambda: (0, 0)),
            pl.BlockSpec((f, c1), lambda: (0, 0)),
            pl.BlockSpec((1, c1), lambda: (0, 0)),
            pl.BlockSpec((c1, c2), lambda: (0, 0)),
            pl.BlockSpec((1, c2), lambda: (0, 0)),
        ],
        out_specs=pl.BlockSpec((n, c2), lambda: (0, 0)),
        out_shape=jax.ShapeDtypeStruct((n, c2), jnp.float32),
        scratch_shapes=[
            pltpu.VMEM((_DEPTH, _BLOCK, k), jnp.float32),
            pltpu.VMEM((kf, c1), jnp.bfloat16),
            pltpu.VMEM((m, c2), jnp.bfloat16),
            pltpu.VMEM((m, c1), jnp.float32),
            pltpu.SemaphoreType.DMA((_DEPTH,)),
        ],
    )(FN, NF, x, W1, b1.reshape(1, -1), W2, b2.reshape(1, -1))


# speedup vs baseline: 1.2329x; 1.0015x over previous
import functools

import jax
import jax.numpy as jnp
from jax.experimental import pallas as pl
from jax.experimental.pallas import tpu as pltpu

_DEPTH = 4
_BLOCK = 256
_QUEUES = 4


def _gcn_body(fn_ref, nf_ref, x_ref, w1_ref, b1_ref, w2_ref, b2_ref,
              out_ref, buf_ref, s1_ref, s2_ref, h_ref, sem_ref):
    m, _ = fn_ref.shape
    n, _ = nf_ref.shape
    nchunk1 = m // _BLOCK
    nchunk2 = n // _BLOCK
    total = nchunk1 + nchunk2

    sub = _BLOCK // _QUEUES

    def copy_in(c, slot):
        c1i = jnp.minimum(c, nchunk1 - 1)
        c2i = jnp.maximum(c - nchunk1, 0)

        def _fn():
            for q in range(_QUEUES):
                pltpu.make_async_copy(
                    fn_ref.at[pl.ds(c1i * _BLOCK + q * sub, sub), :],
                    buf_ref.at[slot, pl.ds(q * sub, sub), :],
                    sem_ref.at[slot, q]).start()

        def _nf():
            for q in range(_QUEUES):
                pltpu.make_async_copy(
                    nf_ref.at[pl.ds(c2i * _BLOCK + q * sub, sub), :],
                    buf_ref.at[slot, pl.ds(q * sub, sub), :],
                    sem_ref.at[slot, q]).start()

        jax.lax.cond(c < nchunk1, _fn, _nf)

    def wait(slot):
        for q in range(_QUEUES):
            pltpu.make_async_copy(
                fn_ref.at[pl.ds(0, sub), :],
                buf_ref.at[slot, pl.ds(q * sub, sub), :],
                sem_ref.at[slot, q]).wait()

    for c in range(_DEPTH):
        copy_in(c, c)
    s1_ref[...] = jnp.dot(x_ref[...], w1_ref[...],
                          preferred_element_type=jnp.float32
                          ).astype(jnp.bfloat16)

    def body1(c, carry):
        slot = jax.lax.rem(c, _DEPTH)
        wait(slot)
        blk = buf_ref[slot].astype(jnp.bfloat16)
        acc = jnp.dot(blk, s1_ref[...], preferred_element_type=jnp.float32)
        h_ref[pl.ds(c * _BLOCK, _BLOCK), :] = jnp.maximum(
            acc + b1_ref[...], 0.0)
        nxt = c + _DEPTH

        @pl.when(nxt < total)
        def _():
            copy_in(nxt, slot)
        return carry

    jax.lax.fori_loop(0, nchunk1, body1, 0)

    s2_ref[...] = jnp.dot(h_ref[...], w2_ref[...],
                          preferred_element_type=jnp.float32
                          ).astype(jnp.bfloat16)

    def body2(c, carry):
        slot = jax.lax.rem(c, _DEPTH)
        wait(slot)
        blk = buf_ref[slot].astype(jnp.bfloat16)
        acc = jnp.dot(blk, s2_ref[...], preferred_element_type=jnp.float32)
        out_ref[pl.ds((c - nchunk1) * _BLOCK, _BLOCK), :] = acc + b2_ref[...]
        nxt = c + _DEPTH

        @pl.when(nxt < total)
        def _():
            copy_in(nxt, slot)
        return carry

    jax.lax.fori_loop(nchunk1, total, body2, 0)


def kernel(x, NF, FN, W1, b1, W2, b2):
    m, k = FN.shape
    n, _ = NF.shape
    kf, f = x.shape
    c1 = W1.shape[1]
    c2 = W2.shape[1]
    return pl.pallas_call(
        _gcn_body,
        in_specs=[
            pl.BlockSpec(memory_space=pltpu.MemorySpace.HBM),
            pl.BlockSpec(memory_space=pltpu.MemorySpace.HBM),
            pl.BlockSpec((kf, f), lambda: (0, 0)),
            pl.BlockSpec((f, c1), lambda: (0, 0)),
            pl.BlockSpec((1, c1), lambda: (0, 0)),
            pl.BlockSpec((c1, c2), lambda: (0, 0)),
            pl.BlockSpec((1, c2), lambda: (0, 0)),
        ],
        out_specs=pl.BlockSpec((n, c2), lambda: (0, 0)),
        out_shape=jax.ShapeDtypeStruct((n, c2), jnp.float32),
        scratch_shapes=[
            pltpu.VMEM((_DEPTH, _BLOCK, k), jnp.float32),
            pltpu.VMEM((kf, c1), jnp.bfloat16),
            pltpu.VMEM((m, c2), jnp.bfloat16),
            pltpu.VMEM((m, c1), jnp.float32),
            pltpu.SemaphoreType.DMA((_DEPTH, _QUEUES)),
        ],
    )(FN, NF, x, W1, b1.reshape(1, -1), W2, b2.reshape(1, -1))
